# baseline (speedup 1.0000x reference)
import functools

import jax
import jax.numpy as jnp
from jax import lax
from jax.experimental import pallas as pl
from jax.experimental.pallas import tpu as pltpu
from jax.experimental.pallas import tpu_sc as plsc

N = 10000
E = 160000
D = 256
H = 4
DH = H * D

NC = 2
NS = 16
NW = NC * NS

RB = 1000
GRID = N // RB

EPT_AP = 5120
E_PAD = NW * EPT_AP
NB_A = EPT_AP // 16

EPT_B = E_PAD // NS
BE = 128
NB_B = EPT_B // BE
CB = 128
NCHUNK = DH // CB
NPASS = NCHUNK // NC


def _dense_body(x_ref, w_ref, al_ref, ar_ref, h_ref, elr_ref, mx_ref):
    i = pl.program_id(0)
    h = jnp.dot(x_ref[...], w_ref[...], preferred_element_type=jnp.float32)
    h_ref[...] = h
    hr = h.reshape(RB, H, D)
    el = (hr * al_ref[...].reshape(1, H, D)).sum(-1)
    er = (hr * ar_ref[...].reshape(1, H, D)).sum(-1)
    elr = jnp.concatenate([el, er], axis=1)
    elr_ref[...] = elr
    m8 = jnp.broadcast_to(jnp.max(elr, axis=0, keepdims=True), (8, 8))

    @pl.when(i == 0)
    def _():
        mx_ref[...] = m8

    @pl.when(i != 0)
    def _():
        mx_ref[...] = jnp.maximum(mx_ref[...], m8)


def _dense(x, w, al, ar):
    return pl.pallas_call(
        _dense_body,
        grid=(GRID,),
        in_specs=[
            pl.BlockSpec((RB, D), lambda i: (i, 0)),
            pl.BlockSpec((D, DH), lambda i: (0, 0)),
            pl.BlockSpec((1, DH), lambda i: (0, 0)),
            pl.BlockSpec((1, DH), lambda i: (0, 0)),
        ],
        out_specs=[
            pl.BlockSpec((RB, DH), lambda i: (i, 0)),
            pl.BlockSpec((RB, 8), lambda i: (i, 0)),
            pl.BlockSpec((8, 8), lambda i: (0, 0)),
        ],
        out_shape=[
            jax.ShapeDtypeStruct((N, DH), jnp.float32),
            jax.ShapeDtypeStruct((N, 8), jnp.float32),
            jax.ShapeDtypeStruct((8, 8), jnp.float32),
        ],
    )(x, w, al, ar)


def _attn_body(ei_hbm, elr_hbm, m_hbm, ex_hbm, tab, sd, exbuf, mbuf):
    w = lax.axis_index("s") * NC + lax.axis_index("c")
    base = w * EPT_AP
    pltpu.sync_copy(elr_hbm, tab)
    pltpu.sync_copy(ei_hbm.at[:, :, pl.ds(base, EPT_AP)], sd)
    pltpu.sync_copy(m_hbm, mbuf)

    mv = mbuf[pl.ds(0, 16)]
    m_l = [jnp.maximum(mv[h] + mv[4 + h], 0.0) for h in range(H)]

    def batch(b, _):
        off = b * 16
        src16 = sd[0, 0, pl.ds(off, 16)] * 8
        dst16 = sd[1, 0, pl.ds(off, 16)] * 8
        valid = (base + off + lax.iota(jnp.int32, 16)) < E
        for h in range(H):
            el = plsc.load_gather(tab, [src16 + h])
            er = plsc.load_gather(tab, [dst16 + (4 + h)])
            e = el + er
            e = jnp.where(e > 0, e, 0.2 * e)
            ex = jnp.exp(e - m_l[h])
            ex = jnp.where(valid, ex, 0.0)
            exbuf[h, 0, pl.ds(off, 16)] = ex
        return 0

    lax.fori_loop(0, NB_A, batch, 0)
    for h in range(H):
        pltpu.sync_copy(exbuf.at[h], ex_hbm.at[h, :, pl.ds(base, EPT_AP)])


def _attn(ei_pad3, elr, m16):
    mesh = plsc.VectorSubcoreMesh(core_axis_name="c", subcore_axis_name="s", num_cores=NC, num_subcores=NS)
    f = functools.partial(
        pl.kernel,
        out_type=jax.ShapeDtypeStruct((H, 1, E_PAD), jnp.float32),
        mesh=mesh,
        compiler_params=pltpu.CompilerParams(needs_layout_passes=False),
        scratch_types=[
            pltpu.VMEM((N * 8,), jnp.float32),
            pltpu.VMEM((2, 1, EPT_AP), jnp.int32),
            pltpu.VMEM((H, 1, EPT_AP), jnp.float32),
            pltpu.VMEM((16,), jnp.float32),
        ],
    )(_attn_body)
    return f(ei_pad3, elr.reshape(N * 8), m16)


def _row_split_copy(copy_fn, s):
    @pl.when(s < 15)
    def _():
        copy_fn(s * 640, 640)

    @pl.when(s == 15)
    def _():
        copy_fn(9600, 400)


def _agg_body(h8_hbm, ex_hbm, ei_hbm, z_hbm, out_hbm,
              o_sh, exb0, exb1, srcb0, srcb1, idxb0, idxb1, dstb0, dstb1,
              gat0, gat1, gsem0, gsem1, ssem0, ssem1):
    c = lax.axis_index("c")
    s = lax.axis_index("s")
    srcb = [srcb0, srcb1]
    idxb = [idxb0, idxb1]
    dstb = [dstb0, dstb1]
    exb = [exb0, exb1]
    gat = [gat0, gat1]
    gsem = [gsem0, gsem1]
    ssem = [ssem0, ssem1]

    def prep(j, par, cchunk):
        base = j * BE
        pltpu.sync_copy(ei_hbm.at[0, s, pl.ds(base, BE)], srcb[par])
        pltpu.sync_copy(ei_hbm.at[1, s, pl.ds(base, BE)], dstb[par])
        for g in range(BE // 16):
            sl = srcb[par][pl.ds(g * 16, 16)]
            idxb[par][pl.ds(g * 16, 16)] = sl * NCHUNK + cchunk

    def prep_ex(j, par, hp):
        pltpu.sync_copy(ex_hbm.at[hp, pl.ds(s * EPT_B + j * BE, BE)], exb[par])

    def gather_start(par):
        pltpu.async_copy(h8_hbm.at[idxb[par]], gat[par], gsem[par])

    def gather_wait(par):
        pltpu.make_async_copy(h8_hbm.at[idxb[par]], gat[par], gsem[par]).wait()

    def scatter_start(par):
        pltpu.async_copy(gat[par], o_sh.at[dstb[par]], ssem[par], add=True)

    def scatter_drain(par):
        pltpu.make_async_copy(h8_hbm.at[idxb[par]], gat[par], ssem[par]).wait()

    for p in range(NPASS):
        hp = 2 * c + (p // 2)
        cchunk = 4 * c + p
        plsc.subcore_barrier()
        _row_split_copy(
            lambda rlo, rows: pltpu.sync_copy(
                z_hbm.at[pl.ds(rlo, rows)],
                o_sh.at[pl.ds(rlo, rows)]), s)
        plsc.subcore_barrier()

        prep(0, 0, cchunk)
        prep_ex(0, 0, hp)
        gather_start(0)

        def step(k, _):
            for par in (0, 1):
                j = 2 * k + par
                gather_wait(par)

                if par == 0:
                    @pl.when(k > 0)
                    def _():
                        scatter_drain(1)

                    prep(j + 1, 1, cchunk)
                    prep_ex(j + 1, 1, hp)
                    gather_start(1)
                else:
                    @pl.when(k < NB_B // 2 - 1)
                    def _():
                        scatter_drain(0)
                        prep(j + 1, 0, cchunk)
                        prep_ex(j + 1, 0, hp)
                        gather_start(0)

                def g16(g, _):
                    exv = exb[par][pl.ds(g * 16, 16)]
                    for u in range(16):
                        i = g * 16 + u
                        exs = exv[u]
                        for q in range(CB // 16):
                            gat[par][i, 0, pl.ds(16 * q, 16)] = (
                                gat[par][i, 0, pl.ds(16 * q, 16)] * exs)
                    return 0

                lax.fori_loop(0, BE // 16, g16, 0)
                scatter_start(par)
            return 0

        lax.fori_loop(0, NB_B // 2, step, 0)
        scatter_drain(0)
        scatter_drain(1)
        plsc.subcore_barrier()
        _row_split_copy(
            lambda rlo, rows: pltpu.sync_copy(
                o_sh.at[pl.ds(rlo, rows)],
                out_hbm.at[pl.ds(rlo, rows), pl.ds(cchunk, 1)]), s)


def _agg(h_mat, exB, ei3, zcol):
    mesh = plsc.VectorSubcoreMesh(core_axis_name="c", subcore_axis_name="s", num_cores=NC, num_subcores=NS)
    f = functools.partial(
        pl.kernel,
        out_type=jax.ShapeDtypeStruct((N, NCHUNK, CB), jnp.float32),
        mesh=mesh,
        compiler_params=pltpu.CompilerParams(needs_layout_passes=False),
        scratch_types=[
            pltpu.VMEM_SHARED((N, 1, CB), jnp.float32),
            pltpu.VMEM((BE,), jnp.float32),
            pltpu.VMEM((BE,), jnp.float32),
            pltpu.VMEM((BE,), jnp.int32),
            pltpu.VMEM((BE,), jnp.int32),
            pltpu.VMEM((BE,), jnp.int32),
            pltpu.VMEM((BE,), jnp.int32),
            pltpu.VMEM((BE,), jnp.int32),
            pltpu.VMEM((BE,), jnp.int32),
            pltpu.VMEM((BE, 1, CB), jnp.float32),
            pltpu.VMEM((BE, 1, CB), jnp.float32),
            pltpu.SemaphoreType.DMA,
            pltpu.SemaphoreType.DMA,
            pltpu.SemaphoreType.DMA,
            pltpu.SemaphoreType.DMA,
        ],
    )(_agg_body)
    pre3 = f(h_mat.reshape(N * NCHUNK, 1, CB), exB, ei3,
             zcol.reshape(N, 1, CB))
    return pre3.reshape(N, DH)


def _sden_body(ei_hbm, ex_hbm, s_hbm, sd, exbuf, s_loc):
    w = lax.axis_index("s") * NC + lax.axis_index("c")
    base = w * EPT_AP
    pltpu.sync_copy(ei_hbm.at[:, :, pl.ds(base, EPT_AP)], sd)
    pltpu.sync_copy(ex_hbm.at[:, :, pl.ds(base, EPT_AP)], exbuf)

    def z16(i, _):
        s_loc[pl.ds(i * 16, 16)] = jnp.zeros((16,), jnp.float32)
        return 0

    lax.fori_loop(0, N * 4 // 16, z16, 0)

    def batch(b, _):
        off = b * 16
        d16 = sd[1, 0, pl.ds(off, 16)] * 4
        for h in range(H):
            exv = exbuf[h, 0, pl.ds(off, 16)]
            plsc.addupdate_scatter(s_loc, [d16 + h], exv)
        return 0

    lax.fori_loop(0, NB_A, batch, 0)
    pltpu.sync_copy(s_loc, s_hbm.at[w])


def _sden(ei_pad3, ex):
    mesh = plsc.VectorSubcoreMesh(core_axis_name="c", subcore_axis_name="s", num_cores=NC, num_subcores=NS)
    f = functools.partial(
        pl.kernel,
        out_type=jax.ShapeDtypeStruct((NW, N * 4), jnp.float32),
        mesh=mesh,
        compiler_params=pltpu.CompilerParams(needs_layout_passes=False),
        scratch_types=[
            pltpu.VMEM((2, 1, EPT_AP), jnp.int32),
            pltpu.VMEM((H, 1, EPT_AP), jnp.float32),
            pltpu.VMEM((N * 4,), jnp.float32),
        ],
    )(_sden_body)
    return f(ei_pad3, ex)


def _post_common(pre_ref, s_ref, b_ref):
    s4 = jnp.sum(s_ref[...], axis=0)
    acc = jnp.zeros((RB, D), jnp.float32)
    for h in range(H):
        sh = s4[:, h:h + 1]
        t = pre_ref[:, D * h:D * (h + 1)] / (sh + 1e-9) + b_ref[0, D * h:D * (h + 1)]
        acc = acc + t
    return acc * 0.25


def _post0_body(pre_ref, s_ref, b_ref, g_ref, bb_ref, out_ref):
    acc = _post_common(pre_ref, s_ref, b_ref)
    mu = acc.mean(-1, keepdims=True)
    var = ((acc - mu) ** 2).mean(-1, keepdims=True)
    y = (acc - mu) / jnp.sqrt(var + 1e-5) * g_ref[0, :] + bb_ref[0, :]
    out_ref[...] = jnp.maximum(y, 0.0)


def _post0(pre, s, b0, ln_g, ln_b):
    return pl.pallas_call(
        _post0_body,
        grid=(GRID,),
        in_specs=[
            pl.BlockSpec((RB, DH), lambda i: (i, 0)),
            pl.BlockSpec((NW, RB, 4), lambda i: (0, i, 0)),
            pl.BlockSpec((1, DH), lambda i: (0, 0)),
            pl.BlockSpec((1, D), lambda i: (0, 0)),
            pl.BlockSpec((1, D), lambda i: (0, 0)),
        ],
        out_specs=pl.BlockSpec((RB, D), lambda i: (i, 0)),
        out_shape=jax.ShapeDtypeStruct((N, D), jnp.float32),
    )(pre, s, b0, ln_g, ln_b)


def _post1_body(pre_ref, s_ref, b_ref, deg_ref, out_ref, fl_ref):
    i = pl.program_id(0)
    out_ref[...] = _post_common(pre_ref, s_ref, b_ref)

    @pl.when(i == 0)
    def _():
        ne = jnp.sum(deg_ref[...]).astype(jnp.float32)
        fl = 2.0 * H * ne * (6.0 * D * D + 6.0 * D + 2.0) / 1e12
        fl_ref[...] = jnp.full((8, 128), fl, jnp.float32)


def _post1(pre, s, b1, in_deg):
    return pl.pallas_call(
        _post1_body,
        grid=(GRID,),
        in_specs=[
            pl.BlockSpec((RB, DH), lambda i: (i, 0)),
            pl.BlockSpec((NW, RB, 4), lambda i: (0, i, 0)),
            pl.BlockSpec((1, DH), lambda i: (0, 0)),
            pl.BlockSpec((8, 1250), lambda i: (0, 0)),
        ],
        out_specs=[
            pl.BlockSpec((RB, D), lambda i: (i, 0)),
            pl.BlockSpec((8, 128), lambda i: (0, 0)),
        ],
        out_shape=[
            jax.ShapeDtypeStruct((N, D), jnp.float32),
            jax.ShapeDtypeStruct((8, 128), jnp.float32),
        ],
    )(pre, s, b1, in_deg)


def _layer(x, ei_pad3, ei3, zcol, W, al, ar):
    h_mat, elr, mx = _dense(x, W, al.reshape(1, DH), ar.reshape(1, DH))
    m16 = jnp.concatenate([mx[0], jnp.zeros((8,), jnp.float32)])
    ex = _attn(ei_pad3, elr, m16)
    pre = _agg(h_mat, ex.reshape(H, E_PAD), ei3, zcol)
    s = _sden(ei_pad3, ex).reshape(NW, N, 4)
    return pre, s


def kernel(feat, edge_index, in_deg, W0, attn_l0, attn_r0, b0, ln_g, ln_b,
           W1, attn_l1, attn_r1, b1):
    ei_pad = jnp.pad(edge_index, ((0, 0), (0, E_PAD - E)))
    ei_pad3 = ei_pad.reshape(2, 1, E_PAD)
    ei3 = ei_pad.reshape(2, NS, EPT_B)
    zcol = jnp.zeros((N, CB), jnp.float32)

    pre0, s0 = _layer(feat, ei_pad3, ei3, zcol, W0, attn_l0, attn_r0)
    h1 = _post0(pre0, s0, b0.reshape(1, DH), ln_g.reshape(1, D), ln_b.reshape(1, D))

    pre1, s1 = _layer(h1, ei_pad3, ei3, zcol, W1, attn_l1, attn_r1)
    out, fl = _post1(pre1, s1, b1.reshape(1, DH), in_deg.reshape(8, 1250))
    return (out, fl[0, 0])


# speedup vs baseline: 1.1436x; 1.1436x over previous
import functools

import jax
import jax.numpy as jnp
from jax import lax
from jax.experimental import pallas as pl
from jax.experimental.pallas import tpu as pltpu
from jax.experimental.pallas import tpu_sc as plsc

N = 10000
E = 160000
D = 256
H = 4
DH = H * D

NC = 2
NS = 16
NW = NC * NS

RB = 1000
GRID = N // RB

EPT_AP = 5120
E_PAD = NW * EPT_AP
NB_A = EPT_AP // 16

EPT_B = E_PAD // NS
BE = 128
NB_B = EPT_B // BE
CB = 128
NCHUNK = DH // CB
NPASS = NCHUNK // NC


def _dense_body(x_ref, w_ref, al_ref, ar_ref, h_ref, elr_ref, mx_ref):
    i = pl.program_id(0)
    h = jnp.dot(x_ref[...], w_ref[...], preferred_element_type=jnp.float32)
    h_ref[...] = h
    hr = h.reshape(RB, H, D)
    el = (hr * al_ref[...].reshape(1, H, D)).sum(-1)
    er = (hr * ar_ref[...].reshape(1, H, D)).sum(-1)
    elr = jnp.concatenate([el, er], axis=1)
    elr_ref[...] = elr
    m8 = jnp.broadcast_to(jnp.max(elr, axis=0, keepdims=True), (8, 8))

    @pl.when(i == 0)
    def _():
        mx_ref[...] = m8

    @pl.when(i != 0)
    def _():
        mx_ref[...] = jnp.maximum(mx_ref[...], m8)


def _dense(x, w, al, ar):
    return pl.pallas_call(
        _dense_body,
        grid=(GRID,),
        in_specs=[
            pl.BlockSpec((RB, D), lambda i: (i, 0)),
            pl.BlockSpec((D, DH), lambda i: (0, 0)),
            pl.BlockSpec((1, DH), lambda i: (0, 0)),
            pl.BlockSpec((1, DH), lambda i: (0, 0)),
        ],
        out_specs=[
            pl.BlockSpec((RB, DH), lambda i: (i, 0)),
            pl.BlockSpec((RB, 8), lambda i: (i, 0)),
            pl.BlockSpec((8, 8), lambda i: (0, 0)),
        ],
        out_shape=[
            jax.ShapeDtypeStruct((N, DH), jnp.float32),
            jax.ShapeDtypeStruct((N, 8), jnp.float32),
            jax.ShapeDtypeStruct((8, 8), jnp.float32),
        ],
    )(x, w, al, ar)


def _attn_body(ei_hbm, elr_hbm, m_hbm, ex_hbm, tab, sd, exbuf, mbuf):
    w = lax.axis_index("s") * NC + lax.axis_index("c")
    base = w * EPT_AP
    pltpu.sync_copy(elr_hbm, tab)
    pltpu.sync_copy(ei_hbm.at[:, :, pl.ds(base, EPT_AP)], sd)
    pltpu.sync_copy(m_hbm, mbuf)

    mv = mbuf[pl.ds(0, 16)]
    m_l = [jnp.maximum(mv[h] + mv[4 + h], 0.0) for h in range(H)]

    def batch(b, _):
        off = b * 16
        src16 = sd[0, 0, pl.ds(off, 16)] * 8
        dst16 = sd[1, 0, pl.ds(off, 16)] * 8
        valid = (base + off + lax.iota(jnp.int32, 16)) < E
        for h in range(H):
            el = plsc.load_gather(tab, [src16 + h])
            er = plsc.load_gather(tab, [dst16 + (4 + h)])
            e = el + er
            e = jnp.where(e > 0, e, 0.2 * e)
            ex = jnp.exp(e - m_l[h])
            ex = jnp.where(valid, ex, 0.0)
            exbuf[h, 0, pl.ds(off, 16)] = ex
        return 0

    lax.fori_loop(0, NB_A, batch, 0)
    for h in range(H):
        pltpu.sync_copy(exbuf.at[h], ex_hbm.at[h, :, pl.ds(base, EPT_AP)])


def _attn(ei_pad3, elr, m16):
    mesh = plsc.VectorSubcoreMesh(core_axis_name="c", subcore_axis_name="s", num_cores=NC, num_subcores=NS)
    f = functools.partial(
        pl.kernel,
        out_type=jax.ShapeDtypeStruct((H, 1, E_PAD), jnp.float32),
        mesh=mesh,
        compiler_params=pltpu.CompilerParams(needs_layout_passes=False),
        scratch_types=[
            pltpu.VMEM((N * 8,), jnp.float32),
            pltpu.VMEM((2, 1, EPT_AP), jnp.int32),
            pltpu.VMEM((H, 1, EPT_AP), jnp.float32),
            pltpu.VMEM((16,), jnp.float32),
        ],
    )(_attn_body)
    return f(ei_pad3, elr.reshape(N * 8), m16)


def _row_split_copy(copy_fn, s):
    @pl.when(s < 15)
    def _():
        copy_fn(s * 640, 640)

    @pl.when(s == 15)
    def _():
        copy_fn(9600, 400)


def _agg_body(h8_hbm, es_hbm, z_hbm, out_hbm,
              o_sh, sdeb0, sdeb1, idxb0, idxb1, dstb0, dstb1,
              gat0, gat1, gsem0, gsem1):
    c = lax.axis_index("c")
    s = lax.axis_index("s")
    sdeb = [sdeb0, sdeb1]
    idxb = [idxb0, idxb1]
    dstb = [dstb0, dstb1]
    gat = [gat0, gat1]
    gsem = [gsem0, gsem1]

    def prep(j, par, cchunk, hp):
        pltpu.sync_copy(es_hbm.at[hp, s, j], sdeb[par])
        for g in range(BE // 16):
            sl = sdeb[par][0, 0, pl.ds(g * 16, 16)]
            idxb[par][pl.ds(g * 16, 16)] = sl * NCHUNK + cchunk
            dstb[par][pl.ds(g * 16, 16)] = sdeb[par][1, 0, pl.ds(g * 16, 16)]

    def prep_ex(j, par, hp):
        pass

    def gather_start(par):
        pltpu.async_copy(h8_hbm.at[idxb[par]], gat[par], gsem[par])

    def gather_wait(par):
        pltpu.make_async_copy(h8_hbm.at[idxb[par]], gat[par], gsem[par]).wait()

    def scatter_start(par):
        pltpu.sync_copy(gat[par], o_sh.at[dstb[par]], add=True)

    def scatter_drain(par):
        pass

    for p in range(NPASS):
        hp = 2 * c + (p // 2)
        cchunk = 4 * c + p
        plsc.subcore_barrier()
        _row_split_copy(
            lambda rlo, rows: pltpu.sync_copy(
                z_hbm.at[pl.ds(rlo, rows)],
                o_sh.at[pl.ds(rlo, rows)]), s)
        plsc.subcore_barrier()

        prep(0, 0, cchunk, hp)
        gather_start(0)

        def step(k, _):
            for par in (0, 1):
                j = 2 * k + par
                gather_wait(par)

                if par == 0:
                    @pl.when(k > 0)
                    def _():
                        scatter_drain(1)

                    prep(j + 1, 1, cchunk, hp)
                    gather_start(1)
                else:
                    @pl.when(k < NB_B // 2 - 1)
                    def _():
                        scatter_drain(0)
                        prep(j + 1, 0, cchunk, hp)
                        gather_start(0)

                def g16(g, _):
                    exv = plsc.bitcast(sdeb[par][2, 0, pl.ds(g * 16, 16)],
                                       jnp.float32)
                    for u in range(16):
                        i = g * 16 + u
                        exs = exv[u]
                        for q in range(CB // 16):
                            gat[par][i, 0, pl.ds(16 * q, 16)] = (
                                gat[par][i, 0, pl.ds(16 * q, 16)] * exs)
                    return 0

                lax.fori_loop(0, BE // 16, g16, 0)
                scatter_start(par)
            return 0

        lax.fori_loop(0, NB_B // 2, step, 0)
        plsc.subcore_barrier()
        _row_split_copy(
            lambda rlo, rows: pltpu.sync_copy(
                o_sh.at[pl.ds(rlo, rows)],
                out_hbm.at[pl.ds(rlo, rows), pl.ds(cchunk, 1)]), s)


def _agg(h_mat, exB, ei3, zcol):
    mesh = plsc.VectorSubcoreMesh(core_axis_name="c", subcore_axis_name="s", num_cores=NC, num_subcores=NS)
    f = functools.partial(
        pl.kernel,
        out_type=jax.ShapeDtypeStruct((N, NCHUNK, CB), jnp.float32),
        mesh=mesh,
        compiler_params=pltpu.CompilerParams(needs_layout_passes=False),
        scratch_types=[
            pltpu.VMEM_SHARED((N, 1, CB), jnp.float32),
            pltpu.VMEM((3, 1, BE), jnp.int32),
            pltpu.VMEM((3, 1, BE), jnp.int32),
            pltpu.VMEM((BE,), jnp.int32),
            pltpu.VMEM((BE,), jnp.int32),
            pltpu.VMEM((BE,), jnp.int32),
            pltpu.VMEM((BE,), jnp.int32),
            pltpu.VMEM((BE, 1, CB), jnp.float32),
            pltpu.VMEM((BE, 1, CB), jnp.float32),
            pltpu.SemaphoreType.DMA,
            pltpu.SemaphoreType.DMA,
        ],
    )(_agg_body)
    pre3 = f(h_mat.reshape(N * NCHUNK, 1, CB), exB,
             zcol.reshape(N, 1, CB))
    return pre3.reshape(N, DH)


def _sden_body(ei_hbm, ex_hbm, s_hbm, sd, exbuf, s_loc):
    w = lax.axis_index("s") * NC + lax.axis_index("c")
    base = w * EPT_AP
    pltpu.sync_copy(ei_hbm.at[:, :, pl.ds(base, EPT_AP)], sd)
    pltpu.sync_copy(ex_hbm.at[:, :, pl.ds(base, EPT_AP)], exbuf)

    def z16(i, _):
        s_loc[pl.ds(i * 16, 16)] = jnp.zeros((16,), jnp.float32)
        return 0

    lax.fori_loop(0, N * 4 // 16, z16, 0)

    def batch(b, _):
        off = b * 16
        d16 = sd[1, 0, pl.ds(off, 16)] * 4
        for h in range(H):
            exv = exbuf[h, 0, pl.ds(off, 16)]
            plsc.addupdate_scatter(s_loc, [d16 + h], exv)
        return 0

    lax.fori_loop(0, NB_A, batch, 0)
    pltpu.sync_copy(s_loc, s_hbm.at[w])


def _sden(ei_pad3, ex):
    mesh = plsc.VectorSubcoreMesh(core_axis_name="c", subcore_axis_name="s", num_cores=NC, num_subcores=NS)
    f = functools.partial(
        pl.kernel,
        out_type=jax.ShapeDtypeStruct((NW, N * 4), jnp.float32),
        mesh=mesh,
        compiler_params=pltpu.CompilerParams(needs_layout_passes=False),
        scratch_types=[
            pltpu.VMEM((2, 1, EPT_AP), jnp.int32),
            pltpu.VMEM((H, 1, EPT_AP), jnp.float32),
            pltpu.VMEM((N * 4,), jnp.float32),
        ],
    )(_sden_body)
    return f(ei_pad3, ex)


def _post_common(pre_ref, s_ref, b_ref):
    s4 = jnp.sum(s_ref[...], axis=0)
    acc = jnp.zeros((RB, D), jnp.float32)
    for h in range(H):
        sh = s4[:, h:h + 1]
        t = pre_ref[:, D * h:D * (h + 1)] / (sh + 1e-9) + b_ref[0, D * h:D * (h + 1)]
        acc = acc + t
    return acc * 0.25


def _post0_body(pre_ref, s_ref, b_ref, g_ref, bb_ref, out_ref):
    acc = _post_common(pre_ref, s_ref, b_ref)
    mu = acc.mean(-1, keepdims=True)
    var = ((acc - mu) ** 2).mean(-1, keepdims=True)
    y = (acc - mu) / jnp.sqrt(var + 1e-5) * g_ref[0, :] + bb_ref[0, :]
    out_ref[...] = jnp.maximum(y, 0.0)


def _post0(pre, s, b0, ln_g, ln_b):
    return pl.pallas_call(
        _post0_body,
        grid=(GRID,),
        in_specs=[
            pl.BlockSpec((RB, DH), lambda i: (i, 0)),
            pl.BlockSpec((NW, RB, 4), lambda i: (0, i, 0)),
            pl.BlockSpec((1, DH), lambda i: (0, 0)),
            pl.BlockSpec((1, D), lambda i: (0, 0)),
            pl.BlockSpec((1, D), lambda i: (0, 0)),
        ],
        out_specs=pl.BlockSpec((RB, D), lambda i: (i, 0)),
        out_shape=jax.ShapeDtypeStruct((N, D), jnp.float32),
    )(pre, s, b0, ln_g, ln_b)


def _post1_body(pre_ref, s_ref, b_ref, deg_ref, out_ref, fl_ref):
    i = pl.program_id(0)
    out_ref[...] = _post_common(pre_ref, s_ref, b_ref)

    @pl.when(i == 0)
    def _():
        ne = jnp.sum(deg_ref[...]).astype(jnp.float32)
        fl = 2.0 * H * ne * (6.0 * D * D + 6.0 * D + 2.0) / 1e12
        fl_ref[...] = jnp.full((8, 128), fl, jnp.float32)


def _post1(pre, s, b1, in_deg):
    return pl.pallas_call(
        _post1_body,
        grid=(GRID,),
        in_specs=[
            pl.BlockSpec((RB, DH), lambda i: (i, 0)),
            pl.BlockSpec((NW, RB, 4), lambda i: (0, i, 0)),
            pl.BlockSpec((1, DH), lambda i: (0, 0)),
            pl.BlockSpec((8, 1250), lambda i: (0, 0)),
        ],
        out_specs=[
            pl.BlockSpec((RB, D), lambda i: (i, 0)),
            pl.BlockSpec((8, 128), lambda i: (0, 0)),
        ],
        out_shape=[
            jax.ShapeDtypeStruct((N, D), jnp.float32),
            jax.ShapeDtypeStruct((8, 128), jnp.float32),
        ],
    )(pre, s, b1, in_deg)


def _layer(x, ei_pad3, ei3, zcol, W, al, ar):
    h_mat, elr, mx = _dense(x, W, al.reshape(1, DH), ar.reshape(1, DH))
    m16 = jnp.concatenate([mx[0], jnp.zeros((8,), jnp.float32)])
    ex = _attn(ei_pad3, elr, m16)
    sd4 = jnp.broadcast_to(ei3.reshape(1, 2, NS, NB_B, BE),
                           (H, 2, NS, NB_B, BE))
    ex4 = jax.lax.bitcast_convert_type(
        ex.reshape(H, NS, NB_B, BE), jnp.int32).reshape(H, 1, NS, NB_B, BE)
    es = jnp.concatenate([sd4, ex4], axis=1)
    es = es.transpose(0, 2, 3, 1, 4).reshape(H, NS, NB_B, 3, 1, BE)
    pre = _agg(h_mat, es, ei3, zcol)
    s = _sden(ei_pad3, ex).reshape(NW, N, 4)
    return pre, s


def kernel(feat, edge_index, in_deg, W0, attn_l0, attn_r0, b0, ln_g, ln_b,
           W1, attn_l1, attn_r1, b1):
    ei_pad = jnp.pad(edge_index, ((0, 0), (0, E_PAD - E)))
    ei_pad3 = ei_pad.reshape(2, 1, E_PAD)
    ei3 = ei_pad.reshape(2, NS, EPT_B)
    zcol = jnp.zeros((N, CB), jnp.float32)

    pre0, s0 = _layer(feat, ei_pad3, ei3, zcol, W0, attn_l0, attn_r0)
    h1 = _post0(pre0, s0, b0.reshape(1, DH), ln_g.reshape(1, D), ln_b.reshape(1, D))

    pre1, s1 = _layer(h1, ei_pad3, ei3, zcol, W1, attn_l1, attn_r1)
    out, fl = _post1(pre1, s1, b1.reshape(1, DH), in_deg.reshape(8, 1250))
    return (out, fl[0, 0])


# speedup vs baseline: 1.2234x; 1.0698x over previous
import functools

import jax
import jax.numpy as jnp
from jax import lax
from jax.experimental import pallas as pl
from jax.experimental.pallas import tpu as pltpu
from jax.experimental.pallas import tpu_sc as plsc

N = 10000
E = 160000
D = 256
H = 4
DH = H * D

NC = 2
NS = 16
NW = NC * NS

RB = 1000
GRID = N // RB

EPT_AP = 5120
E_PAD = NW * EPT_AP
NB_A = EPT_AP // 16

EPT_B = E_PAD // NS
BE = 128
NB_B = EPT_B // BE
CB = 128
NCHUNK = DH // CB
NPASS = NCHUNK // NC


def _dense_body(x_ref, w_ref, al_ref, ar_ref, h_ref, elr_ref, mx_ref):
    i = pl.program_id(0)
    h = jnp.dot(x_ref[...], w_ref[...], preferred_element_type=jnp.float32)
    h_ref[...] = h
    hr = h.reshape(RB, H, D)
    el = (hr * al_ref[...].reshape(1, H, D)).sum(-1)
    er = (hr * ar_ref[...].reshape(1, H, D)).sum(-1)
    elr = jnp.concatenate([el, er], axis=1)
    elr_ref[...] = elr
    m8 = jnp.broadcast_to(jnp.max(elr, axis=0, keepdims=True), (8, 8))

    @pl.when(i == 0)
    def _():
        mx_ref[...] = m8

    @pl.when(i != 0)
    def _():
        mx_ref[...] = jnp.maximum(mx_ref[...], m8)


def _dense(x, w, al, ar):
    return pl.pallas_call(
        _dense_body,
        grid=(GRID,),
        in_specs=[
            pl.BlockSpec((RB, D), lambda i: (i, 0)),
            pl.BlockSpec((D, DH), lambda i: (0, 0)),
            pl.BlockSpec((1, DH), lambda i: (0, 0)),
            pl.BlockSpec((1, DH), lambda i: (0, 0)),
        ],
        out_specs=[
            pl.BlockSpec((RB, DH), lambda i: (i, 0)),
            pl.BlockSpec((RB, 8), lambda i: (i, 0)),
            pl.BlockSpec((8, 8), lambda i: (0, 0)),
        ],
        out_shape=[
            jax.ShapeDtypeStruct((N, DH), jnp.float32),
            jax.ShapeDtypeStruct((N, 8), jnp.float32),
            jax.ShapeDtypeStruct((8, 8), jnp.float32),
        ],
    )(x, w, al, ar)


def _attn_body(ei_hbm, elr_hbm, m_hbm, ex_hbm, tab, sd, exbuf, mbuf):
    w = lax.axis_index("s") * NC + lax.axis_index("c")
    base = w * EPT_AP
    pltpu.sync_copy(elr_hbm, tab)
    pltpu.sync_copy(ei_hbm.at[:, :, pl.ds(base, EPT_AP)], sd)
    pltpu.sync_copy(m_hbm, mbuf)

    mv = mbuf[pl.ds(0, 16)]
    m_l = [jnp.maximum(mv[h] + mv[4 + h], 0.0) for h in range(H)]

    def batch(b, _):
        off = b * 16
        src16 = sd[0, 0, pl.ds(off, 16)] * 8
        dst16 = sd[1, 0, pl.ds(off, 16)] * 8
        valid = (base + off + lax.iota(jnp.int32, 16)) < E
        for h in range(H):
            el = plsc.load_gather(tab, [src16 + h])
            er = plsc.load_gather(tab, [dst16 + (4 + h)])
            e = el + er
            e = jnp.where(e > 0, e, 0.2 * e)
            ex = jnp.exp(e - m_l[h])
            ex = jnp.where(valid, ex, 0.0)
            exbuf[h, 0, pl.ds(off, 16)] = ex
        return 0

    lax.fori_loop(0, NB_A, batch, 0)
    for h in range(H):
        pltpu.sync_copy(exbuf.at[h], ex_hbm.at[h, :, pl.ds(base, EPT_AP)])


def _attn(ei_pad3, elr, m16):
    mesh = plsc.VectorSubcoreMesh(core_axis_name="c", subcore_axis_name="s", num_cores=NC, num_subcores=NS)
    f = functools.partial(
        pl.kernel,
        out_type=jax.ShapeDtypeStruct((H, 1, E_PAD), jnp.float32),
        mesh=mesh,
        compiler_params=pltpu.CompilerParams(needs_layout_passes=False),
        scratch_types=[
            pltpu.VMEM((N * 8,), jnp.float32),
            pltpu.VMEM((2, 1, EPT_AP), jnp.int32),
            pltpu.VMEM((H, 1, EPT_AP), jnp.float32),
            pltpu.VMEM((16,), jnp.float32),
        ],
    )(_attn_body)
    return f(ei_pad3, elr.reshape(N * 8), m16)


def _row_split_copy(copy_fn, s):
    @pl.when(s < 15)
    def _():
        copy_fn(s * 640, 640)

    @pl.when(s == 15)
    def _():
        copy_fn(9600, 400)


def _agg_body(h8_hbm, es_hbm, z_hbm, out_hbm,
              o_sh, sdeb0, sdeb1, idxb0, idxb1, dstb0, dstb1,
              gat0, gat1, gsem0, gsem1):
    c = lax.axis_index("c")
    s = lax.axis_index("s")
    sdeb = [sdeb0, sdeb1]
    idxb = [idxb0, idxb1]
    dstb = [dstb0, dstb1]
    gat = [gat0, gat1]
    gsem = [gsem0, gsem1]

    def prep(j, par, cchunk, hp):
        pltpu.sync_copy(es_hbm.at[hp, s, j], sdeb[par])
        for g in range(BE // 16):
            sl = sdeb[par][0, 0, pl.ds(g * 16, 16)]
            idxb[par][pl.ds(g * 16, 16)] = sl * NCHUNK + cchunk
            dstb[par][pl.ds(g * 16, 16)] = sdeb[par][1, 0, pl.ds(g * 16, 16)]

    def prep_ex(j, par, hp):
        pass

    def gather_start(par):
        pltpu.async_copy(h8_hbm.at[idxb[par]], gat[par], gsem[par])

    def gather_wait(par):
        pltpu.make_async_copy(h8_hbm.at[idxb[par]], gat[par], gsem[par]).wait()

    def scatter_start(par):
        pltpu.sync_copy(gat[par], o_sh.at[dstb[par]], add=True)

    def scatter_drain(par):
        pass

    for p in range(NPASS):
        hp = 2 * c + (p // 2)
        cchunk = 4 * c + p
        plsc.subcore_barrier()
        _row_split_copy(
            lambda rlo, rows: pltpu.sync_copy(
                z_hbm.at[pl.ds(rlo, rows)],
                o_sh.at[pl.ds(rlo, rows)]), s)
        plsc.subcore_barrier()

        prep(0, 0, cchunk, hp)
        gather_start(0)

        def step(k, _):
            for par in (0, 1):
                j = 2 * k + par
                if par == 0:
                    prep(j + 1, 1, cchunk, hp)
                    gather_start(1)
                else:
                    @pl.when(k < NB_B // 2 - 1)
                    def _():
                        prep(j + 1, 0, cchunk, hp)
                        gather_start(0)
                gather_wait(par)

                def g16(g, _):
                    exv = plsc.bitcast(sdeb[par][2, 0, pl.ds(g * 16, 16)],
                                       jnp.float32)
                    for u in range(16):
                        i = g * 16 + u
                        exs = exv[u]
                        for q in range(CB // 16):
                            gat[par][i, 0, pl.ds(16 * q, 16)] = (
                                gat[par][i, 0, pl.ds(16 * q, 16)] * exs)
                    return 0

                lax.fori_loop(0, BE // 16, g16, 0)
                scatter_start(par)
            return 0

        lax.fori_loop(0, NB_B // 2, step, 0)
        plsc.subcore_barrier()
        _row_split_copy(
            lambda rlo, rows: pltpu.sync_copy(
                o_sh.at[pl.ds(rlo, rows)],
                out_hbm.at[pl.ds(rlo, rows), pl.ds(cchunk, 1)]), s)


def _agg(h_mat, exB, ei3, zcol):
    mesh = plsc.VectorSubcoreMesh(core_axis_name="c", subcore_axis_name="s", num_cores=NC, num_subcores=NS)
    f = functools.partial(
        pl.kernel,
        out_type=jax.ShapeDtypeStruct((N, NCHUNK, CB), jnp.float32),
        mesh=mesh,
        compiler_params=pltpu.CompilerParams(needs_layout_passes=False),
        scratch_types=[
            pltpu.VMEM_SHARED((N, 1, CB), jnp.float32),
            pltpu.VMEM((3, 1, BE), jnp.int32),
            pltpu.VMEM((3, 1, BE), jnp.int32),
            pltpu.VMEM((BE,), jnp.int32),
            pltpu.VMEM((BE,), jnp.int32),
            pltpu.VMEM((BE,), jnp.int32),
            pltpu.VMEM((BE,), jnp.int32),
            pltpu.VMEM((BE, 1, CB), jnp.float32),
            pltpu.VMEM((BE, 1, CB), jnp.float32),
            pltpu.SemaphoreType.DMA,
            pltpu.SemaphoreType.DMA,
        ],
    )(_agg_body)
    pre3 = f(h_mat.reshape(N * NCHUNK, 1, CB), exB,
             zcol.reshape(N, 1, CB))
    return pre3.reshape(N, DH)


def _sden_body(ei_hbm, ex_hbm, s_hbm, sd, exbuf, s_loc):
    w = lax.axis_index("s") * NC + lax.axis_index("c")
    base = w * EPT_AP
    pltpu.sync_copy(ei_hbm.at[:, :, pl.ds(base, EPT_AP)], sd)
    pltpu.sync_copy(ex_hbm.at[:, :, pl.ds(base, EPT_AP)], exbuf)

    def z16(i, _):
        s_loc[pl.ds(i * 16, 16)] = jnp.zeros((16,), jnp.float32)
        return 0

    lax.fori_loop(0, N * 4 // 16, z16, 0)

    def batch(b, _):
        off = b * 16
        d16 = sd[1, 0, pl.ds(off, 16)] * 4
        for h in range(H):
            exv = exbuf[h, 0, pl.ds(off, 16)]
            plsc.addupdate_scatter(s_loc, [d16 + h], exv)
        return 0

    lax.fori_loop(0, NB_A, batch, 0)
    pltpu.sync_copy(s_loc, s_hbm.at[w])


def _sden(ei_pad3, ex):
    mesh = plsc.VectorSubcoreMesh(core_axis_name="c", subcore_axis_name="s", num_cores=NC, num_subcores=NS)
    f = functools.partial(
        pl.kernel,
        out_type=jax.ShapeDtypeStruct((NW, N * 4), jnp.float32),
        mesh=mesh,
        compiler_params=pltpu.CompilerParams(needs_layout_passes=False),
        scratch_types=[
            pltpu.VMEM((2, 1, EPT_AP), jnp.int32),
            pltpu.VMEM((H, 1, EPT_AP), jnp.float32),
            pltpu.VMEM((N * 4,), jnp.float32),
        ],
    )(_sden_body)
    return f(ei_pad3, ex)


def _post_common(pre_ref, s_ref, b_ref):
    s4 = jnp.sum(s_ref[...], axis=0)
    acc = jnp.zeros((RB, D), jnp.float32)
    for h in range(H):
        sh = s4[:, h:h + 1]
        t = pre_ref[:, D * h:D * (h + 1)] / (sh + 1e-9) + b_ref[0, D * h:D * (h + 1)]
        acc = acc + t
    return acc * 0.25


def _post0_body(pre_ref, s_ref, b_ref, g_ref, bb_ref, out_ref):
    acc = _post_common(pre_ref, s_ref, b_ref)
    mu = acc.mean(-1, keepdims=True)
    var = ((acc - mu) ** 2).mean(-1, keepdims=True)
    y = (acc - mu) / jnp.sqrt(var + 1e-5) * g_ref[0, :] + bb_ref[0, :]
    out_ref[...] = jnp.maximum(y, 0.0)


def _post0(pre, s, b0, ln_g, ln_b):
    return pl.pallas_call(
        _post0_body,
        grid=(GRID,),
        in_specs=[
            pl.BlockSpec((RB, DH), lambda i: (i, 0)),
            pl.BlockSpec((NW, RB, 4), lambda i: (0, i, 0)),
            pl.BlockSpec((1, DH), lambda i: (0, 0)),
            pl.BlockSpec((1, D), lambda i: (0, 0)),
            pl.BlockSpec((1, D), lambda i: (0, 0)),
        ],
        out_specs=pl.BlockSpec((RB, D), lambda i: (i, 0)),
        out_shape=jax.ShapeDtypeStruct((N, D), jnp.float32),
    )(pre, s, b0, ln_g, ln_b)


def _post1_body(pre_ref, s_ref, b_ref, deg_ref, out_ref, fl_ref):
    i = pl.program_id(0)
    out_ref[...] = _post_common(pre_ref, s_ref, b_ref)

    @pl.when(i == 0)
    def _():
        ne = jnp.sum(deg_ref[...]).astype(jnp.float32)
        fl = 2.0 * H * ne * (6.0 * D * D + 6.0 * D + 2.0) / 1e12
        fl_ref[...] = jnp.full((8, 128), fl, jnp.float32)


def _post1(pre, s, b1, in_deg):
    return pl.pallas_call(
        _post1_body,
        grid=(GRID,),
        in_specs=[
            pl.BlockSpec((RB, DH), lambda i: (i, 0)),
            pl.BlockSpec((NW, RB, 4), lambda i: (0, i, 0)),
            pl.BlockSpec((1, DH), lambda i: (0, 0)),
            pl.BlockSpec((8, 1250), lambda i: (0, 0)),
        ],
        out_specs=[
            pl.BlockSpec((RB, D), lambda i: (i, 0)),
            pl.BlockSpec((8, 128), lambda i: (0, 0)),
        ],
        out_shape=[
            jax.ShapeDtypeStruct((N, D), jnp.float32),
            jax.ShapeDtypeStruct((8, 128), jnp.float32),
        ],
    )(pre, s, b1, in_deg)


def _layer(x, ei_pad3, ei3, zcol, W, al, ar):
    h_mat, elr, mx = _dense(x, W, al.reshape(1, DH), ar.reshape(1, DH))
    m16 = jnp.concatenate([mx[0], jnp.zeros((8,), jnp.float32)])
    ex = _attn(ei_pad3, elr, m16)
    sd4 = jnp.broadcast_to(ei3.reshape(1, 2, NS, NB_B, BE),
                           (H, 2, NS, NB_B, BE))
    ex4 = jax.lax.bitcast_convert_type(
        ex.reshape(H, NS, NB_B, BE), jnp.int32).reshape(H, 1, NS, NB_B, BE)
    es = jnp.concatenate([sd4, ex4], axis=1)
    es = es.transpose(0, 2, 3, 1, 4).reshape(H, NS, NB_B, 3, 1, BE)
    pre = _agg(h_mat, es, ei3, zcol)
    s = _sden(ei_pad3, ex).reshape(NW, N, 4)
    return pre, s


def kernel(feat, edge_index, in_deg, W0, attn_l0, attn_r0, b0, ln_g, ln_b,
           W1, attn_l1, attn_r1, b1):
    ei_pad = jnp.pad(edge_index, ((0, 0), (0, E_PAD - E)))
    ei_pad3 = ei_pad.reshape(2, 1, E_PAD)
    ei3 = ei_pad.reshape(2, NS, EPT_B)
    zcol = jnp.zeros((N, CB), jnp.float32)

    pre0, s0 = _layer(feat, ei_pad3, ei3, zcol, W0, attn_l0, attn_r0)
    h1 = _post0(pre0, s0, b0.reshape(1, DH), ln_g.reshape(1, D), ln_b.reshape(1, D))

    pre1, s1 = _layer(h1, ei_pad3, ei3, zcol, W1, attn_l1, attn_r1)
    out, fl = _post1(pre1, s1, b1.reshape(1, DH), in_deg.reshape(8, 1250))
    return (out, fl[0, 0])
